# Initial kernel scaffold; baseline (speedup 1.0000x reference)
import jax
import jax.numpy as jnp
from jax.experimental import pallas as pl


def kernel(x, W1, b1, W2, b2, Wk1, bk1, Wk2, bk2):
    raise NotImplementedError("write your pallas kernel here")


# speedup vs baseline: 1.2065x; 1.2065x over previous
import functools

import jax
import jax.numpy as jnp
from jax.experimental import pallas as pl
from jax.experimental.pallas import tpu as pltpu


SEQ_BLK = 512


def _stats_body(x_ref, sum_ref, sq_ref):
    b = pl.program_id(0)
    j = pl.program_id(1)
    blk = x_ref[0]
    psum = jnp.sum(blk, axis=0)
    psq = jnp.sum(blk * blk, axis=0)

    @pl.when(j == 0)
    def _init():
        sum_ref[b, :] = psum
        sq_ref[b, :] = psq

    @pl.when(j != 0)
    def _acc():
        sum_ref[b, :] = sum_ref[b, :] + psum
        sq_ref[b, :] = sq_ref[b, :] + psq


def _gate_body(scale, sum_ref, sq_ref, wk1_ref, bk1_ref, wk2_ref, bk2_ref,
               k_ref, mask_ref):
    gs = jnp.sum(sum_ref[...], axis=0, keepdims=True) * scale
    h = jax.lax.dot_general(gs, wk1_ref[...], (((1,), (1,)), ((), ())),
                            preferred_element_type=jnp.float32)
    h = jnp.maximum(h + bk1_ref[...], 0.0)
    logit = jnp.sum(h * wk2_ref[...], axis=1, keepdims=True)
    k = jax.nn.sigmoid(logit + bk2_ref[0])
    k_ref[...] = jnp.clip(k, 0.3, 1.0)

    norms = jnp.sqrt(sq_ref[...])
    mu = jnp.mean(norms, axis=-1, keepdims=True)
    mask_ref[...] = jax.nn.sigmoid((norms - mu) * 10.0)


def _mul_body(x_ref, m_ref, o_ref):
    b = pl.program_id(0)
    o_ref[0] = x_ref[0] * m_ref[b, :]


def kernel(x, W1, b1, W2, b2, Wk1, bk1, Wk2, bk2):
    batch, seq, d = x.shape
    nsb = seq // SEQ_BLK

    sums, sqs = pl.pallas_call(
        _stats_body,
        grid=(batch, nsb),
        in_specs=[pl.BlockSpec((1, SEQ_BLK, d), lambda b, j: (b, j, 0))],
        out_specs=[pl.BlockSpec((batch, d), lambda b, j: (0, 0)),
                   pl.BlockSpec((batch, d), lambda b, j: (0, 0))],
        out_shape=[jax.ShapeDtypeStruct((batch, d), jnp.float32),
                   jax.ShapeDtypeStruct((batch, d), jnp.float32)],
    )(x)

    k2, mask = pl.pallas_call(
        functools.partial(_gate_body, 1.0 / (batch * seq)),
        in_specs=[pl.BlockSpec(s.shape, lambda: (0, 0))
                  for s in (sums, sqs, Wk1, bk1.reshape(1, -1), Wk2)]
        + [pl.BlockSpec(memory_space=pltpu.SMEM)],
        out_specs=[pl.BlockSpec((1, 1), lambda: (0, 0)),
                   pl.BlockSpec((batch, d), lambda: (0, 0))],
        out_shape=[jax.ShapeDtypeStruct((1, 1), jnp.float32),
                   jax.ShapeDtypeStruct((batch, d), jnp.float32)],
    )(sums, sqs, Wk1, bk1.reshape(1, -1), Wk2, bk2)

    pruned = pl.pallas_call(
        _mul_body,
        grid=(batch, nsb),
        in_specs=[pl.BlockSpec((1, SEQ_BLK, d), lambda b, j: (b, j, 0)),
                  pl.BlockSpec((batch, d), lambda b, j: (0, 0))],
        out_specs=pl.BlockSpec((1, SEQ_BLK, d), lambda b, j: (b, j, 0)),
        out_shape=jax.ShapeDtypeStruct((batch, seq, d), jnp.float32),
    )(x, mask)

    return pruned, k2.reshape(())
